# baseline (speedup 1.0000x reference)
import functools

import jax
import jax.numpy as jnp
from jax import lax
from jax.experimental import pallas as pl
from jax.experimental.pallas import tpu as pltpu
from jax.experimental.pallas import tpu_sc as plsc

N = 10000
E = 320000
IN_DIM = 128
HID = 256
NUM_MASK = 3000
NPAD = 10240
NC, NS = 2, 16
K = 128
CH1 = 80
CH2 = 160
EPAD = NC * NS * CH1 * K
NACC = 10112
RPS = NACC // NS
RB = 512
GRID = NPAD // RB


def _make_spmm(split, ch):
    mesh = plsc.VectorSubcoreMesh(core_axis_name="c", subcore_axis_name="s")

    @functools.partial(
        pl.kernel,
        out_type=jax.ShapeDtypeStruct((NC, NPAD, 128), jnp.float32),
        mesh=mesh,
        scratch_types=[
            pltpu.VMEM((2, K), jnp.int32),
            pltpu.VMEM((2, K), jnp.int32),
            pltpu.VMEM((K, 128), jnp.float32),
            pltpu.VMEM((K, 128), jnp.float32),
            pltpu.VMEM_SHARED((NACC, 128), jnp.float32),
            pltpu.SemaphoreType.DMA,
            pltpu.SemaphoreType.DMA,
            pltpu.SemaphoreType.DMA,
            pltpu.SemaphoreType.DMA,
            pltpu.SemaphoreType.DMA,
            pltpu.SemaphoreType.DMA,
        ],
    )
    def spmm(table, idx, zeros, out, ia, ib, rows_a, rows_b, acc,
             sem_ia, sem_ib, sem_ga, sem_gb, sem_sa, sem_sb):
        c = lax.axis_index("c")
        s = lax.axis_index("s")
        r0 = s * RPS
        if split == "edge":
            ip = idx.at[c * NS + s]
        else:
            ip = idx.at[c, s]
        pltpu.sync_copy(zeros.at[pl.ds(r0, RPS)], acc.at[pl.ds(r0, RPS)])
        plsc.subcore_barrier()

        def load_idx(g, buf, sem):
            pltpu.async_copy(ip.at[pl.ds(2 * g, 2)], buf, sem)

        def wait_idx(buf, sem):
            pltpu.make_async_copy(ip.at[pl.ds(0, 2)], buf, sem).wait()

        def gather(buf, rows, sem):
            pltpu.async_copy(table.at[buf.at[0]], rows, sem)

        def wait_gather(rows, sem):
            pltpu.make_async_copy(table.at[ia.at[0]], rows, sem).wait()

        def scat_async(buf, rows, sem):
            pltpu.async_copy(rows, acc.at[buf.at[1]], sem, add=True)

        def wait_scat(buf, rows, sem):
            pltpu.make_async_copy(rows, acc.at[buf.at[1]], sem).wait()

        load_idx(0, ia, sem_ia)
        load_idx(1, ib, sem_ib)
        wait_idx(ia, sem_ia)
        gather(ia, rows_a, sem_ga)
        wait_idx(ib, sem_ib)
        gather(ib, rows_b, sem_gb)

        def refill(buf, rows, sem_i, sem_g, sem_s, g_next):
            @pl.when(g_next < ch)
            def _():
                wait_scat(buf, rows, sem_s)
                load_idx(g_next, buf, sem_i)
                wait_idx(buf, sem_i)
                gather(buf, rows, sem_g)

        def pair(t, carry):
            g0 = 2 * t
            wait_gather(rows_a, sem_ga)
            scat_async(ia, rows_a, sem_sa)
            wait_gather(rows_b, sem_gb)
            scat_async(ib, rows_b, sem_sb)
            refill(ia, rows_a, sem_ia, sem_ga, sem_sa, g0 + 2)
            refill(ib, rows_b, sem_ib, sem_gb, sem_sb, g0 + 3)
            return carry

        lax.fori_loop(0, ch // 2, pair, 0)
        wait_scat(ia, rows_a, sem_sa)
        wait_scat(ib, rows_b, sem_sb)
        plsc.subcore_barrier()
        pltpu.sync_copy(acc.at[pl.ds(r0, RPS)], out.at[c, pl.ds(r0, RPS)])

    return spmm


def _make_deg():
    mesh = plsc.VectorSubcoreMesh(core_axis_name="c", subcore_axis_name="s")

    @functools.partial(
        pl.kernel,
        out_type=jax.ShapeDtypeStruct((NC, NPAD, 16), jnp.float32),
        mesh=mesh,
        scratch_types=[
            pltpu.VMEM((2, K), jnp.int32),
            pltpu.VMEM((2, K), jnp.int32),
            pltpu.VMEM((2, K), jnp.int32),
            pltpu.VMEM((2, K), jnp.int32),
            pltpu.VMEM((K, 16), jnp.float32),
            pltpu.VMEM_SHARED((NACC, 16), jnp.float32),
            pltpu.SemaphoreType.DMA,
            pltpu.SemaphoreType.DMA,
            pltpu.SemaphoreType.DMA,
            pltpu.SemaphoreType.DMA,
            pltpu.SemaphoreType.DMA,
            pltpu.SemaphoreType.DMA,
            pltpu.SemaphoreType.DMA,
            pltpu.SemaphoreType.DMA,
        ],
    )
    def deg(idx, ones16, zeros16, out, ia, ib, ic, id_, ones_v, acc,
            si_a, si_b, si_c, si_d, ss_a, ss_b, ss_c, ss_d):
        c = lax.axis_index("c")
        s = lax.axis_index("s")
        r0 = s * RPS
        ip = idx.at[c * NS + s]
        bufs = (ia, ib, ic, id_)
        isems = (si_a, si_b, si_c, si_d)
        ssems = (ss_a, ss_b, ss_c, ss_d)

        def load_idx(g, buf, sem):
            pltpu.async_copy(ip.at[pl.ds(2 * g, 2)], buf, sem)

        def wait_idx(buf, sem):
            pltpu.make_async_copy(ip.at[pl.ds(0, 2)], buf, sem).wait()

        def wait_scat(buf, sem):
            pltpu.make_async_copy(ones_v, acc.at[buf.at[1]], sem).wait()

        for x in range(4):
            load_idx(x, bufs[x], isems[x])
        pltpu.sync_copy(ones16, ones_v)
        pltpu.sync_copy(zeros16.at[pl.ds(r0, RPS)], acc.at[pl.ds(r0, RPS)])
        plsc.subcore_barrier()

        def quad(t, carry):
            g0 = 4 * t
            for x in range(4):
                wait_idx(bufs[x], isems[x])
                pltpu.async_copy(ones_v, acc.at[bufs[x].at[1]], ssems[x],
                                 add=True)
            for x in range(4):
                @pl.when(g0 + 4 + x < CH1)
                def _(x=x):
                    wait_scat(bufs[x], ssems[x])
                    load_idx(g0 + 4 + x, bufs[x], isems[x])
            return carry

        lax.fori_loop(0, CH1 // 4, quad, 0)
        for x in range(4):
            wait_scat(bufs[x], ssems[x])
        plsc.subcore_barrier()
        pltpu.sync_copy(acc.at[pl.ds(r0, RPS)], out.at[c, pl.ds(r0, RPS)])

    return deg


def _rowspec(shape3=False, minor=128):
    if shape3:
        return pl.BlockSpec((NC, RB, minor), lambda i: (0, i, 0))
    return pl.BlockSpec((RB, minor), lambda i: (i, 0))


def _fullspec(shape):
    nd = len(shape)
    return pl.BlockSpec(shape, lambda i: (0,) * nd)


def _valid(i):
    row = i * RB + lax.broadcasted_iota(jnp.int32, (RB, 128), 0)
    return row < N


def _t0_body(deg_ref, x_ref, m_ref, tok_ref, dn_ref, x1_ref):
    d = deg_ref[0, :, :1] + deg_ref[1, :, :1] + 1.0
    dn = jnp.broadcast_to(lax.rsqrt(d), (RB, 128))
    m = m_ref[...]
    use_x = m * tok_ref[...] + (1.0 - m) * x_ref[...]
    dn_ref[...] = dn
    x1_ref[...] = jnp.where(_valid(pl.program_id(0)), dn * use_x, 0.0)


def _t0(degp, xp, maskf, tok):
    return pl.pallas_call(
        _t0_body,
        grid=(GRID,),
        in_specs=[_rowspec(True, minor=16), _rowspec(), _rowspec(),
                  _fullspec((1, 128))],
        out_specs=[_rowspec(), _rowspec()],
        out_shape=[jax.ShapeDtypeStruct((NPAD, 128), jnp.float32),
                   jax.ShapeDtypeStruct((NPAD, 128), jnp.float32)],
    )(degp, xp, maskf, tok)


def _t1_body(aggp_ref, x1_ref, dn_ref, w1_ref, b1_ref, a1_ref, out_ref):
    pre = dn_ref[...] * (aggp_ref[0] + aggp_ref[1] + x1_ref[...])
    h = jnp.dot(pre, w1_ref[...], preferred_element_type=jnp.float32)
    h = h + b1_ref[...]
    h = jnp.where(h >= 0.0, h, a1_ref[...] * h)
    dnw = jnp.concatenate([dn_ref[...], dn_ref[...]], axis=1)
    v = _valid(pl.program_id(0))
    x2 = jnp.where(jnp.concatenate([v, v], axis=1), dnw * h, 0.0)
    out_ref[0] = x2[:, :128]
    out_ref[1] = x2[:, 128:]


def _t1(agg1p, x1, dn, W1, b1r, a1r):
    return pl.pallas_call(
        _t1_body,
        grid=(GRID,),
        in_specs=[_rowspec(True), _rowspec(), _rowspec(),
                  _fullspec((IN_DIM, HID)), _fullspec((1, HID)),
                  _fullspec((1, HID))],
        out_specs=_rowspec(True),
        out_shape=jax.ShapeDtypeStruct((NC, NPAD, 128), jnp.float32),
    )(agg1p, x1, dn, W1, b1r, a1r)


def _t2_body(aggh_ref, x2_ref, dn_ref, w2_ref, b2_ref, a2_ref, wf_ref,
             m_ref, out_ref):
    agg = jnp.concatenate([aggh_ref[0], aggh_ref[1]], axis=1)
    x2 = jnp.concatenate([x2_ref[0], x2_ref[1]], axis=1)
    dnw = jnp.concatenate([dn_ref[...], dn_ref[...]], axis=1)
    pre = dnw * (agg + x2)
    h = jnp.dot(pre, w2_ref[...], preferred_element_type=jnp.float32)
    h = h + b2_ref[...]
    enc = jnp.where(h >= 0.0, h, a2_ref[...] * h)
    z = jnp.dot(enc, wf_ref[...], preferred_element_type=jnp.float32)
    z = (1.0 - m_ref[...]) * z
    out_ref[...] = jnp.where(_valid(pl.program_id(0)),
                             dn_ref[...] * z, 0.0)


def _t2(agg2h, x2, dn, W2, b2r, a2r, wf, maskf):
    return pl.pallas_call(
        _t2_body,
        grid=(GRID,),
        in_specs=[_rowspec(True), _rowspec(True), _rowspec(),
                  _fullspec((HID, HID)), _fullspec((1, HID)),
                  _fullspec((1, HID)), _fullspec((HID, 128)), _rowspec()],
        out_specs=_rowspec(),
        out_shape=jax.ShapeDtypeStruct((NPAD, 128), jnp.float32),
    )(agg2h, x2, dn, W2, b2r, a2r, wf, maskf)


def _t3_body(aggp_ref, x3_ref, dn_ref, bd_ref, x_ref, m_ref, out_ref):
    i = pl.program_id(0)

    @pl.when(i == 0)
    def _():
        out_ref[...] = jnp.zeros_like(out_ref)

    recon = dn_ref[...] * (aggp_ref[0] + aggp_ref[1] + x3_ref[...])
    recon = recon + bd_ref[...]
    xb = x_ref[...]
    dot = jnp.sum(recon * xb, axis=1)
    nr = jnp.maximum(jnp.sqrt(jnp.sum(recon * recon, axis=1)), 1e-8)
    nx = jnp.maximum(jnp.sqrt(jnp.sum(xb * xb, axis=1)), 1e-8)
    cos = dot / (nr * nx)
    contrib = jnp.where(m_ref[:, 0] > 0.0,
                        (1.0 - cos) * (1.0 - cos), 0.0)
    part = jnp.sum(contrib) * (1.0 / NUM_MASK)
    out_ref[...] = out_ref[...] + jnp.full((8, 128), part, jnp.float32)


def _t3(agg3p, x3, dn, bdr, xp, maskf):
    return pl.pallas_call(
        _t3_body,
        grid=(GRID,),
        in_specs=[_rowspec(True), _rowspec(), _rowspec(),
                  _fullspec((1, 128)), _rowspec(), _rowspec()],
        out_specs=_fullspec((8, 128)),
        out_shape=jax.ShapeDtypeStruct((8, 128), jnp.float32),
    )(agg3p, x3, dn, bdr, xp, maskf)


def _wf_body(wed_ref, wdec_ref, out_ref):
    out_ref[...] = jnp.dot(wed_ref[...], wdec_ref[...],
                           preferred_element_type=jnp.float32)


def _wfuse(Wed, Wdec):
    return pl.pallas_call(
        _wf_body,
        in_specs=[pl.BlockSpec((HID, HID), lambda: (0, 0)),
                  pl.BlockSpec((HID, IN_DIM), lambda: (0, 0))],
        out_specs=pl.BlockSpec((HID, IN_DIM), lambda: (0, 0)),
        out_shape=jax.ShapeDtypeStruct((HID, IN_DIM), jnp.float32),
    )(Wed, Wdec)


def kernel(x, edge_index, W1, b1, a1, W2, b2, a2, Wed, Wdec, bdec, mask_token):
    with jax.ensure_compile_time_eval():
        perm = jax.random.permutation(jax.random.key(123), N)
        mask_flag = jnp.zeros((N,), bool).at[perm[:NUM_MASK]].set(True)
        mf = jnp.zeros((NPAD,), jnp.float32).at[:N].set(
            mask_flag.astype(jnp.float32))
        maskf = jnp.broadcast_to(mf[:, None], (NPAD, 128))
        ones_tab = jnp.ones((NPAD, 128), jnp.float32)
        zeros_tab = jnp.zeros((NPAD, 128), jnp.float32)
        pad_n = EPAD - E
        pad_src = (jnp.arange(pad_n, dtype=jnp.int32) * 613) % N
        pad_dst = N + (jnp.arange(pad_n, dtype=jnp.int32) % (NPAD - N))

    src_p = jnp.concatenate([edge_index[0], pad_src])
    dst_p = jnp.concatenate([edge_index[1], pad_dst])
    src1 = src_p.reshape(NC * NS, CH1, K)
    dst1 = dst_p.reshape(NC * NS, CH1, K)
    src_a = src_p.reshape(NS, CH2, K)
    src2 = jnp.stack([src_a, src_a + NPAD])
    dst2 = dst_p.reshape(NS, CH2, K)

    spmm_e = _make_spmm("edge", CH1)
    spmm_f = _make_spmm("feature", CH2)

    degp = _make_deg()(pair1, jnp.ones((K, 16), jnp.float32),
                       jnp.zeros((NPAD, 16), jnp.float32))
    dn, x1 = _t0(degp, x, maskf, mask_token)

    agg1p = spmm_e(x1, pair1, zeros_tab)
    x2 = _t1(agg1p, x1, dn, W1, b1[None, :], jnp.broadcast_to(a1, (1, HID)))

    agg2h = spmm_f(x2.reshape(2 * NPAD, 128), pair2, zeros_tab)
    wf = _wfuse(Wed, Wdec)
    x3 = _t2(agg2h, x2, dn, W2, b2[None, :],
             jnp.broadcast_to(a2, (1, HID)), wf, maskf)

    agg3p = spmm_e(x3, pair1, zeros_tab)
    losst = _t3(agg3p, x3, dn, bdec[None, :], x, maskf)
    return losst[0, 0]


# speedup vs baseline: 1.0700x; 1.0700x over previous
import functools

import jax
import jax.numpy as jnp
from jax import lax
from jax.experimental import pallas as pl
from jax.experimental.pallas import tpu as pltpu
from jax.experimental.pallas import tpu_sc as plsc

N = 10000
E = 320000
IN_DIM = 128
HID = 256
NUM_MASK = 3000
NPAD = 10240
NC, NS = 2, 16
K = 128
CH1 = 80
CH2 = 160
EPAD = NC * NS * CH1 * K
NACC = 10112
RPS = NACC // NS
RB = 512
GRID = NPAD // RB


def _make_spmm(split, ch):
    mesh = plsc.VectorSubcoreMesh(core_axis_name="c", subcore_axis_name="s")

    @functools.partial(
        pl.kernel,
        out_type=jax.ShapeDtypeStruct((NC, NPAD, 128), jnp.float32),
        mesh=mesh,
        scratch_types=[
            pltpu.VMEM((2, K), jnp.int32),
            pltpu.VMEM((2, K), jnp.int32),
            pltpu.VMEM((K, 128), jnp.float32),
            pltpu.VMEM((K, 128), jnp.float32),
            pltpu.VMEM_SHARED((NACC, 128), jnp.float32),
            pltpu.SemaphoreType.DMA,
            pltpu.SemaphoreType.DMA,
            pltpu.SemaphoreType.DMA,
            pltpu.SemaphoreType.DMA,
        ],
    )
    def spmm(table, idx, zeros, out, ia, ib, rows_a, rows_b, acc,
             sem_ia, sem_ib, sem_ga, sem_gb):
        c = lax.axis_index("c")
        s = lax.axis_index("s")
        r0 = s * RPS
        if split == "edge":
            ip = idx.at[c * NS + s]
        else:
            ip = idx.at[c, s]
        pltpu.sync_copy(zeros.at[pl.ds(r0, RPS)], acc.at[pl.ds(r0, RPS)])
        plsc.subcore_barrier()

        def load_idx(g, buf, sem):
            pltpu.async_copy(ip.at[pl.ds(2 * g, 2)], buf, sem)

        def wait_idx(buf, sem):
            pltpu.make_async_copy(ip.at[pl.ds(0, 2)], buf, sem).wait()

        def gather(buf, rows, sem):
            pltpu.async_copy(table.at[buf.at[0]], rows, sem)

        def wait_gather(rows, sem):
            pltpu.make_async_copy(table.at[ia.at[0]], rows, sem).wait()

        def scat(buf, rows):
            pltpu.sync_copy(rows, acc.at[buf.at[1]], add=True)

        load_idx(0, ia, sem_ia)
        load_idx(1, ib, sem_ib)
        wait_idx(ia, sem_ia)
        gather(ia, rows_a, sem_ga)

        def stage(buf, rows, sem_i, sem_g, g_next):
            wait_gather(rows, sem_g)
            scat(buf, rows)

            @pl.when(g_next < ch)
            def _():
                load_idx(g_next, buf, sem_i)
                wait_idx(buf, sem_i)
                gather(buf, rows, sem_g)

        def pair(t, carry):
            g0 = 2 * t
            wait_idx(ib, sem_ib)
            gather(ib, rows_b, sem_gb)
            stage(ia, rows_a, sem_ia, sem_ga, g0 + 2)
            wait_gather(rows_b, sem_gb)
            scat(ib, rows_b)

            @pl.when(g0 + 3 < ch)
            def _():
                load_idx(g0 + 3, ib, sem_ib)

            return carry

        lax.fori_loop(0, ch // 2, pair, 0)
        plsc.subcore_barrier()
        pltpu.sync_copy(acc.at[pl.ds(r0, RPS)], out.at[c, pl.ds(r0, RPS)])

    return spmm


def _make_deg():
    mesh = plsc.VectorSubcoreMesh(core_axis_name="c", subcore_axis_name="s")

    @functools.partial(
        pl.kernel,
        out_type=jax.ShapeDtypeStruct((NC, NPAD, 16), jnp.float32),
        mesh=mesh,
        scratch_types=[
            pltpu.VMEM((2, K), jnp.int32),
            pltpu.VMEM((2, K), jnp.int32),
            pltpu.VMEM((2, K), jnp.int32),
            pltpu.VMEM((2, K), jnp.int32),
            pltpu.VMEM((K, 16), jnp.float32),
            pltpu.VMEM_SHARED((NACC, 16), jnp.float32),
            pltpu.SemaphoreType.DMA,
            pltpu.SemaphoreType.DMA,
            pltpu.SemaphoreType.DMA,
            pltpu.SemaphoreType.DMA,
            pltpu.SemaphoreType.DMA,
            pltpu.SemaphoreType.DMA,
            pltpu.SemaphoreType.DMA,
            pltpu.SemaphoreType.DMA,
        ],
    )
    def deg(idx, ones16, zeros16, out, ia, ib, ic, id_, ones_v, acc,
            si_a, si_b, si_c, si_d, ss_a, ss_b, ss_c, ss_d):
        c = lax.axis_index("c")
        s = lax.axis_index("s")
        r0 = s * RPS
        ip = idx.at[c * NS + s]
        bufs = (ia, ib, ic, id_)
        isems = (si_a, si_b, si_c, si_d)
        ssems = (ss_a, ss_b, ss_c, ss_d)

        def load_idx(g, buf, sem):
            pltpu.async_copy(ip.at[pl.ds(2 * g, 2)], buf, sem)

        def wait_idx(buf, sem):
            pltpu.make_async_copy(ip.at[pl.ds(0, 2)], buf, sem).wait()

        def wait_scat(buf, sem):
            pltpu.make_async_copy(ones_v, acc.at[buf.at[1]], sem).wait()

        for x in range(4):
            load_idx(x, bufs[x], isems[x])
        pltpu.sync_copy(ones16, ones_v)
        pltpu.sync_copy(zeros16.at[pl.ds(r0, RPS)], acc.at[pl.ds(r0, RPS)])
        plsc.subcore_barrier()

        def quad(t, carry):
            g0 = 4 * t
            for x in range(4):
                wait_idx(bufs[x], isems[x])
                pltpu.async_copy(ones_v, acc.at[bufs[x].at[1]], ssems[x],
                                 add=True)
            for x in range(4):
                @pl.when(g0 + 4 + x < CH1)
                def _(x=x):
                    wait_scat(bufs[x], ssems[x])
                    load_idx(g0 + 4 + x, bufs[x], isems[x])
            return carry

        lax.fori_loop(0, CH1 // 4, quad, 0)
        for x in range(4):
            wait_scat(bufs[x], ssems[x])
        plsc.subcore_barrier()
        pltpu.sync_copy(acc.at[pl.ds(r0, RPS)], out.at[c, pl.ds(r0, RPS)])

    return deg


def _rowspec(shape3=False, minor=128):
    if shape3:
        return pl.BlockSpec((NC, RB, minor), lambda i: (0, i, 0))
    return pl.BlockSpec((RB, minor), lambda i: (i, 0))


def _fullspec(shape):
    nd = len(shape)
    return pl.BlockSpec(shape, lambda i: (0,) * nd)


def _valid(i):
    row = i * RB + lax.broadcasted_iota(jnp.int32, (RB, 128), 0)
    return row < N


def _t0_body(deg_ref, x_ref, m_ref, tok_ref, dn_ref, x1_ref):
    d = deg_ref[0, :, :1] + deg_ref[1, :, :1] + 1.0
    dn = jnp.broadcast_to(lax.rsqrt(d), (RB, 128))
    m = m_ref[...]
    use_x = m * tok_ref[...] + (1.0 - m) * x_ref[...]
    dn_ref[...] = dn
    x1_ref[...] = jnp.where(_valid(pl.program_id(0)), dn * use_x, 0.0)


def _t0(degp, xp, maskf, tok):
    return pl.pallas_call(
        _t0_body,
        grid=(GRID,),
        in_specs=[_rowspec(True, minor=16), _rowspec(), _rowspec(),
                  _fullspec((1, 128))],
        out_specs=[_rowspec(), _rowspec()],
        out_shape=[jax.ShapeDtypeStruct((NPAD, 128), jnp.float32),
                   jax.ShapeDtypeStruct((NPAD, 128), jnp.float32)],
    )(degp, xp, maskf, tok)


def _t1_body(aggp_ref, x1_ref, dn_ref, w1_ref, b1_ref, a1_ref, out_ref):
    pre = dn_ref[...] * (aggp_ref[0] + aggp_ref[1] + x1_ref[...])
    h = jnp.dot(pre, w1_ref[...], preferred_element_type=jnp.float32)
    h = h + b1_ref[...]
    h = jnp.where(h >= 0.0, h, a1_ref[...] * h)
    dnw = jnp.concatenate([dn_ref[...], dn_ref[...]], axis=1)
    v = _valid(pl.program_id(0))
    x2 = jnp.where(jnp.concatenate([v, v], axis=1), dnw * h, 0.0)
    out_ref[0] = x2[:, :128]
    out_ref[1] = x2[:, 128:]


def _t1(agg1p, x1, dn, W1, b1r, a1r):
    return pl.pallas_call(
        _t1_body,
        grid=(GRID,),
        in_specs=[_rowspec(True), _rowspec(), _rowspec(),
                  _fullspec((IN_DIM, HID)), _fullspec((1, HID)),
                  _fullspec((1, HID))],
        out_specs=_rowspec(True),
        out_shape=jax.ShapeDtypeStruct((NC, NPAD, 128), jnp.float32),
    )(agg1p, x1, dn, W1, b1r, a1r)


def _t2_body(aggh_ref, x2_ref, dn_ref, w2_ref, b2_ref, a2_ref, wf_ref,
             m_ref, out_ref):
    agg = jnp.concatenate([aggh_ref[0], aggh_ref[1]], axis=1)
    x2 = jnp.concatenate([x2_ref[0], x2_ref[1]], axis=1)
    dnw = jnp.concatenate([dn_ref[...], dn_ref[...]], axis=1)
    pre = dnw * (agg + x2)
    h = jnp.dot(pre, w2_ref[...], preferred_element_type=jnp.float32)
    h = h + b2_ref[...]
    enc = jnp.where(h >= 0.0, h, a2_ref[...] * h)
    z = jnp.dot(enc, wf_ref[...], preferred_element_type=jnp.float32)
    z = (1.0 - m_ref[...]) * z
    out_ref[...] = jnp.where(_valid(pl.program_id(0)),
                             dn_ref[...] * z, 0.0)


def _t2(agg2h, x2, dn, W2, b2r, a2r, wf, maskf):
    return pl.pallas_call(
        _t2_body,
        grid=(GRID,),
        in_specs=[_rowspec(True), _rowspec(True), _rowspec(),
                  _fullspec((HID, HID)), _fullspec((1, HID)),
                  _fullspec((1, HID)), _fullspec((HID, 128)), _rowspec()],
        out_specs=_rowspec(),
        out_shape=jax.ShapeDtypeStruct((NPAD, 128), jnp.float32),
    )(agg2h, x2, dn, W2, b2r, a2r, wf, maskf)


def _t3_body(aggp_ref, x3_ref, dn_ref, bd_ref, x_ref, m_ref, out_ref):
    i = pl.program_id(0)

    @pl.when(i == 0)
    def _():
        out_ref[...] = jnp.zeros_like(out_ref)

    recon = dn_ref[...] * (aggp_ref[0] + aggp_ref[1] + x3_ref[...])
    recon = recon + bd_ref[...]
    xb = x_ref[...]
    dot = jnp.sum(recon * xb, axis=1)
    nr = jnp.maximum(jnp.sqrt(jnp.sum(recon * recon, axis=1)), 1e-8)
    nx = jnp.maximum(jnp.sqrt(jnp.sum(xb * xb, axis=1)), 1e-8)
    cos = dot / (nr * nx)
    contrib = jnp.where(m_ref[:, 0] > 0.0,
                        (1.0 - cos) * (1.0 - cos), 0.0)
    part = jnp.sum(contrib) * (1.0 / NUM_MASK)
    out_ref[...] = out_ref[...] + jnp.full((8, 128), part, jnp.float32)


def _t3(agg3p, x3, dn, bdr, xp, maskf):
    return pl.pallas_call(
        _t3_body,
        grid=(GRID,),
        in_specs=[_rowspec(True), _rowspec(), _rowspec(),
                  _fullspec((1, 128)), _rowspec(), _rowspec()],
        out_specs=_fullspec((8, 128)),
        out_shape=jax.ShapeDtypeStruct((8, 128), jnp.float32),
    )(agg3p, x3, dn, bdr, xp, maskf)


def _wf_body(wed_ref, wdec_ref, out_ref):
    out_ref[...] = jnp.dot(wed_ref[...], wdec_ref[...],
                           preferred_element_type=jnp.float32)


def _wfuse(Wed, Wdec):
    return pl.pallas_call(
        _wf_body,
        in_specs=[pl.BlockSpec((HID, HID), lambda: (0, 0)),
                  pl.BlockSpec((HID, IN_DIM), lambda: (0, 0))],
        out_specs=pl.BlockSpec((HID, IN_DIM), lambda: (0, 0)),
        out_shape=jax.ShapeDtypeStruct((HID, IN_DIM), jnp.float32),
    )(Wed, Wdec)


def kernel(x, edge_index, W1, b1, a1, W2, b2, a2, Wed, Wdec, bdec, mask_token):
    with jax.ensure_compile_time_eval():
        perm = jax.random.permutation(jax.random.key(123), N)
        mask_flag = jnp.zeros((N,), bool).at[perm[:NUM_MASK]].set(True)
        mf = jnp.zeros((NPAD,), jnp.float32).at[:N].set(
            mask_flag.astype(jnp.float32))
        maskf = jnp.broadcast_to(mf[:, None], (NPAD, 128))
        ones_tab = jnp.ones((NPAD, 128), jnp.float32)
        zeros_tab = jnp.zeros((NPAD, 128), jnp.float32)
        pad_n = EPAD - E
        pad_src = (jnp.arange(pad_n, dtype=jnp.int32) * 613) % N
        pad_dst = N + (jnp.arange(pad_n, dtype=jnp.int32) % (NPAD - N))

    src_p = jnp.concatenate([edge_index[0], pad_src])
    dst_p = jnp.concatenate([edge_index[1], pad_dst])
    src1 = src_p.reshape(NC * NS, CH1, K)
    dst1 = dst_p.reshape(NC * NS, CH1, K)
    src_a = src_p.reshape(NS, CH2, K)
    src2 = jnp.stack([src_a, src_a + NPAD])
    dst2 = dst_p.reshape(NS, CH2, K)

    spmm_e = _make_spmm("edge", CH1)
    spmm_f = _make_spmm("feature", CH2)

    degp = _make_deg()(pair1, jnp.ones((K, 16), jnp.float32),
                       jnp.zeros((NPAD, 16), jnp.float32))
    dn, x1 = _t0(degp, x, maskf, mask_token)

    agg1p = spmm_e(x1, pair1, zeros_tab)
    x2 = _t1(agg1p, x1, dn, W1, b1[None, :], jnp.broadcast_to(a1, (1, HID)))

    agg2h = spmm_f(x2.reshape(2 * NPAD, 128), pair2, zeros_tab)
    wf = _wfuse(Wed, Wdec)
    x3 = _t2(agg2h, x2, dn, W2, b2[None, :],
             jnp.broadcast_to(a2, (1, HID)), wf, maskf)

    agg3p = spmm_e(x3, pair1, zeros_tab)
    losst = _t3(agg3p, x3, dn, bdec[None, :], x, maskf)
    return losst[0, 0]


# speedup vs baseline: 1.0814x; 1.0106x over previous
import functools

import jax
import jax.numpy as jnp
from jax import lax
from jax.experimental import pallas as pl
from jax.experimental.pallas import tpu as pltpu
from jax.experimental.pallas import tpu_sc as plsc

N = 10000
E = 320000
IN_DIM = 128
HID = 256
NUM_MASK = 3000
NPAD = 10240
NC, NS = 2, 16
K = 128
CH1 = 80
CH2 = 160
EPAD = NC * NS * CH1 * K
NACC = 10112
RPS = NACC // NS
RB = 512
GRID = NPAD // RB


def _make_spmm(split, ch):
    mesh = plsc.VectorSubcoreMesh(core_axis_name="c", subcore_axis_name="s")

    @functools.partial(
        pl.kernel,
        out_type=jax.ShapeDtypeStruct((NC, NPAD, 128), jnp.float32),
        mesh=mesh,
        scratch_types=[
            pltpu.VMEM((2, K), jnp.int32),
            pltpu.VMEM((2, K), jnp.int32),
            pltpu.VMEM((K, 128), jnp.float32),
            pltpu.VMEM((K, 128), jnp.float32),
            pltpu.VMEM_SHARED((NACC, 128), jnp.float32),
            pltpu.SemaphoreType.DMA,
            pltpu.SemaphoreType.DMA,
            pltpu.SemaphoreType.DMA,
            pltpu.SemaphoreType.DMA,
            pltpu.SemaphoreType.DMA,
        ],
    )
    def spmm(table, idx, zeros, out, ia, ib, rows_a, rows_b, acc,
             sem_ia, sem_ib, sem_ga, sem_gb, sem_z):
        c = lax.axis_index("c")
        s = lax.axis_index("s")
        r0 = s * RPS
        if split == "edge":
            ip = idx.at[c * NS + s]
        else:
            ip = idx.at[c, s]

        def load_idx(g, buf, sem):
            pltpu.async_copy(ip.at[pl.ds(2 * g, 2)], buf, sem)

        def wait_idx(buf, sem):
            pltpu.make_async_copy(ip.at[pl.ds(0, 2)], buf, sem).wait()

        def gather(buf, rows, sem):
            pltpu.async_copy(table.at[buf.at[0]], rows, sem)

        def wait_gather(rows, sem):
            pltpu.make_async_copy(table.at[ia.at[0]], rows, sem).wait()

        def scat(buf, rows):
            pltpu.sync_copy(rows, acc.at[buf.at[1]], add=True)

        pltpu.async_copy(zeros.at[pl.ds(r0, RPS)], acc.at[pl.ds(r0, RPS)],
                         sem_z)
        load_idx(0, ia, sem_ia)
        load_idx(1, ib, sem_ib)
        wait_idx(ia, sem_ia)
        gather(ia, rows_a, sem_ga)
        pltpu.make_async_copy(zeros.at[pl.ds(r0, RPS)],
                              acc.at[pl.ds(r0, RPS)], sem_z).wait()
        plsc.subcore_barrier()

        def stage(buf, rows, sem_i, sem_g, g_next):
            wait_gather(rows, sem_g)
            scat(buf, rows)

            @pl.when(g_next < ch)
            def _():
                load_idx(g_next, buf, sem_i)
                wait_idx(buf, sem_i)
                gather(buf, rows, sem_g)

        def pair(t, carry):
            g0 = 2 * t
            wait_idx(ib, sem_ib)
            gather(ib, rows_b, sem_gb)
            stage(ia, rows_a, sem_ia, sem_ga, g0 + 2)
            wait_gather(rows_b, sem_gb)
            scat(ib, rows_b)

            @pl.when(g0 + 3 < ch)
            def _():
                load_idx(g0 + 3, ib, sem_ib)

            return carry

        lax.fori_loop(0, ch // 2, pair, 0)
        plsc.subcore_barrier()
        pltpu.sync_copy(acc.at[pl.ds(r0, RPS)], out.at[c, pl.ds(r0, RPS)])

    return spmm


def _make_deg():
    mesh = plsc.VectorSubcoreMesh(core_axis_name="c", subcore_axis_name="s")

    @functools.partial(
        pl.kernel,
        out_type=jax.ShapeDtypeStruct((NC, NPAD, 16), jnp.float32),
        mesh=mesh,
        scratch_types=[
            pltpu.VMEM((2, K), jnp.int32),
            pltpu.VMEM((2, K), jnp.int32),
            pltpu.VMEM((2, K), jnp.int32),
            pltpu.VMEM((2, K), jnp.int32),
            pltpu.VMEM((K, 16), jnp.float32),
            pltpu.VMEM_SHARED((NACC, 16), jnp.float32),
            pltpu.SemaphoreType.DMA,
            pltpu.SemaphoreType.DMA,
            pltpu.SemaphoreType.DMA,
            pltpu.SemaphoreType.DMA,
            pltpu.SemaphoreType.DMA,
            pltpu.SemaphoreType.DMA,
            pltpu.SemaphoreType.DMA,
            pltpu.SemaphoreType.DMA,
        ],
    )
    def deg(idx, ones16, zeros16, out, ia, ib, ic, id_, ones_v, acc,
            si_a, si_b, si_c, si_d, ss_a, ss_b, ss_c, ss_d):
        c = lax.axis_index("c")
        s = lax.axis_index("s")
        r0 = s * RPS
        ip = idx.at[c * NS + s]
        bufs = (ia, ib, ic, id_)
        isems = (si_a, si_b, si_c, si_d)
        ssems = (ss_a, ss_b, ss_c, ss_d)

        def load_idx(g, buf, sem):
            pltpu.async_copy(ip.at[pl.ds(2 * g, 2)], buf, sem)

        def wait_idx(buf, sem):
            pltpu.make_async_copy(ip.at[pl.ds(0, 2)], buf, sem).wait()

        def wait_scat(buf, sem):
            pltpu.make_async_copy(ones_v, acc.at[buf.at[1]], sem).wait()

        for x in range(4):
            load_idx(x, bufs[x], isems[x])
        pltpu.sync_copy(ones16, ones_v)
        pltpu.sync_copy(zeros16.at[pl.ds(r0, RPS)], acc.at[pl.ds(r0, RPS)])
        plsc.subcore_barrier()

        def quad(t, carry):
            g0 = 4 * t
            for x in range(4):
                wait_idx(bufs[x], isems[x])
                pltpu.async_copy(ones_v, acc.at[bufs[x].at[1]], ssems[x],
                                 add=True)
            for x in range(4):
                @pl.when(g0 + 4 + x < CH1)
                def _(x=x):
                    wait_scat(bufs[x], ssems[x])
                    load_idx(g0 + 4 + x, bufs[x], isems[x])
            return carry

        lax.fori_loop(0, CH1 // 4, quad, 0)
        for x in range(4):
            wait_scat(bufs[x], ssems[x])
        plsc.subcore_barrier()
        pltpu.sync_copy(acc.at[pl.ds(r0, RPS)], out.at[c, pl.ds(r0, RPS)])

    return deg


def _rowspec(shape3=False, minor=128):
    if shape3:
        return pl.BlockSpec((NC, RB, minor), lambda i: (0, i, 0))
    return pl.BlockSpec((RB, minor), lambda i: (i, 0))


def _fullspec(shape):
    nd = len(shape)
    return pl.BlockSpec(shape, lambda i: (0,) * nd)


def _valid(i):
    row = i * RB + lax.broadcasted_iota(jnp.int32, (RB, 128), 0)
    return row < N


def _t0_body(deg_ref, x_ref, m_ref, tok_ref, dn_ref, x1_ref):
    d = deg_ref[0, :, :1] + deg_ref[1, :, :1] + 1.0
    dn = jnp.broadcast_to(lax.rsqrt(d), (RB, 128))
    m = m_ref[...]
    use_x = m * tok_ref[...] + (1.0 - m) * x_ref[...]
    dn_ref[...] = dn
    x1_ref[...] = jnp.where(_valid(pl.program_id(0)), dn * use_x, 0.0)


def _t0(degp, xp, maskf, tok):
    return pl.pallas_call(
        _t0_body,
        grid=(GRID,),
        in_specs=[_rowspec(True, minor=16), _rowspec(), _rowspec(),
                  _fullspec((1, 128))],
        out_specs=[_rowspec(), _rowspec()],
        out_shape=[jax.ShapeDtypeStruct((NPAD, 128), jnp.float32),
                   jax.ShapeDtypeStruct((NPAD, 128), jnp.float32)],
    )(degp, xp, maskf, tok)


def _t1_body(aggp_ref, x1_ref, dn_ref, w1_ref, b1_ref, a1_ref, out_ref):
    pre = dn_ref[...] * (aggp_ref[0] + aggp_ref[1] + x1_ref[...])
    h = jnp.dot(pre, w1_ref[...], preferred_element_type=jnp.float32)
    h = h + b1_ref[...]
    h = jnp.where(h >= 0.0, h, a1_ref[...] * h)
    dnw = jnp.concatenate([dn_ref[...], dn_ref[...]], axis=1)
    v = _valid(pl.program_id(0))
    x2 = jnp.where(jnp.concatenate([v, v], axis=1), dnw * h, 0.0)
    out_ref[0] = x2[:, :128]
    out_ref[1] = x2[:, 128:]


def _t1(agg1p, x1, dn, W1, b1r, a1r):
    return pl.pallas_call(
        _t1_body,
        grid=(GRID,),
        in_specs=[_rowspec(True), _rowspec(), _rowspec(),
                  _fullspec((IN_DIM, HID)), _fullspec((1, HID)),
                  _fullspec((1, HID))],
        out_specs=_rowspec(True),
        out_shape=jax.ShapeDtypeStruct((NC, NPAD, 128), jnp.float32),
    )(agg1p, x1, dn, W1, b1r, a1r)


def _t2_body(aggh_ref, x2_ref, dn_ref, w2_ref, b2_ref, a2_ref, wf_ref,
             m_ref, out_ref):
    agg = jnp.concatenate([aggh_ref[0], aggh_ref[1]], axis=1)
    x2 = jnp.concatenate([x2_ref[0], x2_ref[1]], axis=1)
    dnw = jnp.concatenate([dn_ref[...], dn_ref[...]], axis=1)
    pre = dnw * (agg + x2)
    h = jnp.dot(pre, w2_ref[...], preferred_element_type=jnp.float32)
    h = h + b2_ref[...]
    enc = jnp.where(h >= 0.0, h, a2_ref[...] * h)
    z = jnp.dot(enc, wf_ref[...], preferred_element_type=jnp.float32)
    z = (1.0 - m_ref[...]) * z
    out_ref[...] = jnp.where(_valid(pl.program_id(0)),
                             dn_ref[...] * z, 0.0)


def _t2(agg2h, x2, dn, W2, b2r, a2r, wf, maskf):
    return pl.pallas_call(
        _t2_body,
        grid=(GRID,),
        in_specs=[_rowspec(True), _rowspec(True), _rowspec(),
                  _fullspec((HID, HID)), _fullspec((1, HID)),
                  _fullspec((1, HID)), _fullspec((HID, 128)), _rowspec()],
        out_specs=_rowspec(),
        out_shape=jax.ShapeDtypeStruct((NPAD, 128), jnp.float32),
    )(agg2h, x2, dn, W2, b2r, a2r, wf, maskf)


def _t3_body(aggp_ref, x3_ref, dn_ref, bd_ref, x_ref, m_ref, out_ref):
    i = pl.program_id(0)

    @pl.when(i == 0)
    def _():
        out_ref[...] = jnp.zeros_like(out_ref)

    recon = dn_ref[...] * (aggp_ref[0] + aggp_ref[1] + x3_ref[...])
    recon = recon + bd_ref[...]
    xb = x_ref[...]
    dot = jnp.sum(recon * xb, axis=1)
    nr = jnp.maximum(jnp.sqrt(jnp.sum(recon * recon, axis=1)), 1e-8)
    nx = jnp.maximum(jnp.sqrt(jnp.sum(xb * xb, axis=1)), 1e-8)
    cos = dot / (nr * nx)
    contrib = jnp.where(m_ref[:, 0] > 0.0,
                        (1.0 - cos) * (1.0 - cos), 0.0)
    part = jnp.sum(contrib) * (1.0 / NUM_MASK)
    out_ref[...] = out_ref[...] + jnp.full((8, 128), part, jnp.float32)


def _t3(agg3p, x3, dn, bdr, xp, maskf):
    return pl.pallas_call(
        _t3_body,
        grid=(GRID,),
        in_specs=[_rowspec(True), _rowspec(), _rowspec(),
                  _fullspec((1, 128)), _rowspec(), _rowspec()],
        out_specs=_fullspec((8, 128)),
        out_shape=jax.ShapeDtypeStruct((8, 128), jnp.float32),
    )(agg3p, x3, dn, bdr, xp, maskf)


def _wf_body(wed_ref, wdec_ref, out_ref):
    out_ref[...] = jnp.dot(wed_ref[...], wdec_ref[...],
                           preferred_element_type=jnp.float32)


def _wfuse(Wed, Wdec):
    return pl.pallas_call(
        _wf_body,
        in_specs=[pl.BlockSpec((HID, HID), lambda: (0, 0)),
                  pl.BlockSpec((HID, IN_DIM), lambda: (0, 0))],
        out_specs=pl.BlockSpec((HID, IN_DIM), lambda: (0, 0)),
        out_shape=jax.ShapeDtypeStruct((HID, IN_DIM), jnp.float32),
    )(Wed, Wdec)


def kernel(x, edge_index, W1, b1, a1, W2, b2, a2, Wed, Wdec, bdec, mask_token):
    with jax.ensure_compile_time_eval():
        perm = jax.random.permutation(jax.random.key(123), N)
        mask_flag = jnp.zeros((N,), bool).at[perm[:NUM_MASK]].set(True)
        mf = jnp.zeros((NPAD,), jnp.float32).at[:N].set(
            mask_flag.astype(jnp.float32))
        maskf = jnp.broadcast_to(mf[:, None], (NPAD, 128))
        ones_tab = jnp.ones((NPAD, 128), jnp.float32)
        zeros_tab = jnp.zeros((NPAD, 128), jnp.float32)
        pad_n = EPAD - E
        pad_src = (jnp.arange(pad_n, dtype=jnp.int32) * 613) % N
        pad_dst = N + (jnp.arange(pad_n, dtype=jnp.int32) % (NPAD - N))

    src_p = jnp.concatenate([edge_index[0], pad_src])
    dst_p = jnp.concatenate([edge_index[1], pad_dst])
    src1 = src_p.reshape(NC * NS, CH1, K)
    dst1 = dst_p.reshape(NC * NS, CH1, K)
    src_a = src_p.reshape(NS, CH2, K)
    src2 = jnp.stack([src_a, src_a + NPAD])
    dst2 = dst_p.reshape(NS, CH2, K)

    spmm_e = _make_spmm("edge", CH1)
    spmm_f = _make_spmm("feature", CH2)

    degp = _make_deg()(pair1, jnp.ones((K, 16), jnp.float32),
                       jnp.zeros((NPAD, 16), jnp.float32))
    dn, x1 = _t0(degp, x, maskf, mask_token)

    agg1p = spmm_e(x1, pair1, zeros_tab)
    x2 = _t1(agg1p, x1, dn, W1, b1[None, :], jnp.broadcast_to(a1, (1, HID)))

    agg2h = spmm_f(x2.reshape(2 * NPAD, 128), pair2, zeros_tab)
    wf = _wfuse(Wed, Wdec)
    x3 = _t2(agg2h, x2, dn, W2, b2[None, :],
             jnp.broadcast_to(a2, (1, HID)), wf, maskf)

    agg3p = spmm_e(x3, pair1, zeros_tab)
    losst = _t3(agg3p, x3, dn, bdec[None, :], x, maskf)
    return losst[0, 0]
